# Initial kernel scaffold; baseline (speedup 1.0000x reference)
import jax
import jax.numpy as jnp
from jax.experimental import pallas as pl


def kernel(x, edge_index, params, sage):
    raise NotImplementedError("write your pallas kernel here")


# speedup vs baseline: 10.1541x; 10.1541x over previous
import functools

import jax
import jax.numpy as jnp
from jax import lax
from jax.experimental import pallas as pl
from jax.experimental.pallas import tpu as pltpu
from jax.experimental.pallas import tpu_sc as plsc

_EPS = 1e-7
_N = 50000
_E = 800000
_N_PAD = 51200
_NW = 32
_CHUNK = 128
_N_CHUNKS = 196
_E_PAD = _NW * _N_CHUNKS * _CHUNK
_N_PAIRS = _N_CHUNKS // 2
_ROWS_PER_TILE = _N_PAD // 16
_Z_CHUNKS = _ROWS_PER_TILE // _CHUNK
_BN = 1024
_GRID = _N_PAD // _BN

_LAYER_BLOCKS = [(1, 16), (2, 16), (3, 16), (5, 16), (9, 16), (5, 16), (3, 16)]
_DIMS = [(5, 8), (13, 16), (21, 32), (37, 64), (69, 32), (37, 16), (21, 8)]


@functools.cache
def _sc_agg(nblk, w):
    mesh = plsc.VectorSubcoreMesh(core_axis_name="c", subcore_axis_name="s")

    def body(src_hbm, dst_hbm, *rest):
        tables = rest[:nblk]
        out_hbm = rest[nblk]
        idx_s, idx_d, rows, zbuf, acc, sg0, sg1 = rest[nblk + 1:]
        cid = lax.axis_index("c")
        sid = lax.axis_index("s")
        wid = cid * 16 + sid
        pltpu.sync_copy(src_hbm.at[wid], idx_s)
        pltpu.sync_copy(dst_hbm.at[wid], idx_d)
        z16 = jnp.zeros((16,), jnp.float32)

        def zrow(r, carry):
            for c in range(w // 16):
                zbuf[r, pl.ds(c * 16, 16)] = z16
            return carry

        lax.fori_loop(0, _CHUNK, zrow, 0)
        rbase = sid * _ROWS_PER_TILE

        def zero_acc():
            def zb(j, carry):
                pltpu.sync_copy(zbuf, acc.at[pl.ds(rbase + j * _CHUNK, _CHUNK)])
                return carry

            lax.fori_loop(0, _Z_CHUNKS, zb, 0)

        zero_acc()
        for b in range(nblk):
            tab = tables[b]
            plsc.subcore_barrier()
            pltpu.async_copy(tab.at[idx_s.at[0]], rows.at[0], sg0)

            def pair(j, carry):
                i0 = 2 * j
                pltpu.make_async_copy(tab.at[idx_s.at[i0]], rows.at[0],
                                      sg0).wait()
                pltpu.async_copy(tab.at[idx_s.at[i0 + 1]], rows.at[1], sg1)
                pltpu.sync_copy(rows.at[0], acc.at[idx_d.at[i0]], add=True)
                pltpu.make_async_copy(tab.at[idx_s.at[i0 + 1]], rows.at[1],
                                      sg1).wait()

                @pl.when(j + 1 < _N_PAIRS)
                def _():
                    pltpu.async_copy(tab.at[idx_s.at[i0 + 2]], rows.at[0], sg0)

                pltpu.sync_copy(rows.at[1], acc.at[idx_d.at[i0 + 1]], add=True)
                return carry

            lax.fori_loop(0, _N_PAIRS, pair, 0)
            plsc.subcore_barrier()
            pltpu.sync_copy(
                acc.at[pl.ds(rbase, _ROWS_PER_TILE)],
                out_hbm.at[b, cid, pl.ds(rbase, _ROWS_PER_TILE)])
            if b + 1 < nblk:
                zero_acc()

    return pl.kernel(
        body,
        out_type=jax.ShapeDtypeStruct((nblk, 2, _N_PAD, w), jnp.float32),
        mesh=mesh,
        scratch_types=[
            pltpu.VMEM((_N_CHUNKS, _CHUNK), jnp.int32),
            pltpu.VMEM((_N_CHUNKS, _CHUNK), jnp.int32),
            pltpu.VMEM((2, _CHUNK, w), jnp.float32),
            pltpu.VMEM((_CHUNK, w), jnp.float32),
            pltpu.VMEM_SHARED((_N_PAD, w), jnp.float32),
            pltpu.SemaphoreType.DMA,
            pltpu.SemaphoreType.DMA,
        ],
        compiler_params=pltpu.CompilerParams(use_tc_tiling_on_sc=False),
        name=f"sc_edge_agg_{nblk}x{w}",
    )


@functools.cache
def _tc_max0(c):
    def body(h_ref, t_ref, smax_ref):
        s = t_ref[0, 0] * (jnp.maximum(h_ref[...], 0.0) + _EPS)
        local = jnp.max(s, axis=0, keepdims=True)
        i = pl.program_id(0)

        @pl.when(i == 0)
        def _():
            smax_ref[...] = local

        @pl.when(i > 0)
        def _():
            smax_ref[...] = jnp.maximum(smax_ref[...], local)

    return pl.pallas_call(
        body,
        grid=(_GRID,),
        in_specs=[
            pl.BlockSpec((_BN, c), lambda i: (i, 0)),
            pl.BlockSpec((1, 1), lambda i: (0, 0)),
        ],
        out_specs=pl.BlockSpec((1, c), lambda i: (0, 0)),
        out_shape=jax.ShapeDtypeStruct((1, c), jnp.float32),
    )


@functools.cache
def _tc_table(c, nblk, w):
    def body(h_ref, smax_ref, t_ref, *out_refs):
        h = h_ref[...]
        msg = jnp.maximum(h, 0.0) + _EPS
        s = t_ref[0, 0] * msg
        e = jnp.exp(s - smax_ref[...])
        full = jnp.concatenate([e, e * msg], axis=1)
        padw = nblk * w - 2 * c
        if padw:
            full = jnp.concatenate(
                [full, jnp.zeros((_BN, padw), jnp.float32)], axis=1)
        for b in range(nblk):
            out_refs[b][...] = full[:, b * w:(b + 1) * w]

    return pl.pallas_call(
        body,
        grid=(_GRID,),
        in_specs=[
            pl.BlockSpec((_BN, c), lambda i: (i, 0)),
            pl.BlockSpec((1, c), lambda i: (0, 0)),
            pl.BlockSpec((1, 1), lambda i: (0, 0)),
        ],
        out_specs=[pl.BlockSpec((_BN, w), lambda i: (i, 0))] * nblk,
        out_shape=[jax.ShapeDtypeStruct((_N_PAD, w), jnp.float32)] * nblk,
    )


@functools.cache
def _tc_post(c, cout, nblk, w):
    hdim = 2 * c
    cn = cout + 5

    def body(acc_ref, h_ref, x0_ref, w1_ref, b1_ref, g_ref, be_ref, w2_ref,
             b2_ref, tn_ref, hcat_ref, smax_ref):
        blocks = [acc_ref[b, 0] + acc_ref[b, 1] for b in range(nblk)]
        full = jnp.concatenate(blocks, axis=1) if nblk > 1 else blocks[0]
        den = full[:, :c]
        num = full[:, c:2 * c]
        agg = num / jnp.maximum(den, 1e-16)
        out = agg + h_ref[...]
        hm = jnp.dot(out, w1_ref[...], preferred_element_type=jnp.float32)
        hm = (hm + b1_ref[...]) * g_ref[...] + be_ref[...]
        hm = jnp.maximum(hm, 0.0)
        h2 = jnp.dot(hm, w2_ref[...], preferred_element_type=jnp.float32)
        h2 = h2 + b2_ref[...]
        hcat = jnp.concatenate([h2, x0_ref[...]], axis=1)
        i = pl.program_id(0)
        rows = i * _BN + lax.broadcasted_iota(jnp.int32, (_BN, 1), 0)
        hcat = jnp.where(rows < _N, hcat, 0.0)
        hcat_ref[...] = hcat
        s = tn_ref[0, 0] * (jnp.maximum(hcat, 0.0) + _EPS)
        local = jnp.max(s, axis=0, keepdims=True)

        @pl.when(i == 0)
        def _():
            smax_ref[...] = local

        @pl.when(i > 0)
        def _():
            smax_ref[...] = jnp.maximum(smax_ref[...], local)

    return pl.pallas_call(
        body,
        grid=(_GRID,),
        in_specs=[
            pl.BlockSpec((nblk, 2, _BN, w), lambda i: (0, 0, i, 0)),
            pl.BlockSpec((_BN, c), lambda i: (i, 0)),
            pl.BlockSpec((_BN, 5), lambda i: (i, 0)),
            pl.BlockSpec((c, hdim), lambda i: (0, 0)),
            pl.BlockSpec((1, hdim), lambda i: (0, 0)),
            pl.BlockSpec((1, hdim), lambda i: (0, 0)),
            pl.BlockSpec((1, hdim), lambda i: (0, 0)),
            pl.BlockSpec((hdim, cout), lambda i: (0, 0)),
            pl.BlockSpec((1, cout), lambda i: (0, 0)),
            pl.BlockSpec((1, 1), lambda i: (0, 0)),
        ],
        out_specs=[
            pl.BlockSpec((_BN, cn), lambda i: (i, 0)),
            pl.BlockSpec((1, cn), lambda i: (0, 0)),
        ],
        out_shape=[
            jax.ShapeDtypeStruct((_N_PAD, cn), jnp.float32),
            jax.ShapeDtypeStruct((1, cn), jnp.float32),
        ],
    )


@functools.cache
def _tc_sage_table(c, w):
    def body(h_ref, out_ref):
        h = h_ref[...]
        ones = jnp.ones((_BN, 1), jnp.float32)
        pad = jnp.zeros((_BN, w - c - 1), jnp.float32)
        out_ref[...] = jnp.concatenate([h, ones, pad], axis=1)

    return pl.pallas_call(
        body,
        grid=(_GRID,),
        in_specs=[pl.BlockSpec((_BN, c), lambda i: (i, 0))],
        out_specs=pl.BlockSpec((_BN, w), lambda i: (i, 0)),
        out_shape=jax.ShapeDtypeStruct((_N_PAD, w), jnp.float32),
    )


@functools.cache
def _tc_final(c, w):
    def body(acc_ref, h_ref, wl_ref, bl_ref, wr_ref, y_ref):
        a = acc_ref[0, 0] + acc_ref[0, 1]
        ssum = a[:, :c]
        deg = a[:, c:c + 1]
        mean = ssum / jnp.maximum(deg, 1.0)
        y = jnp.dot(mean, wl_ref[...], preferred_element_type=jnp.float32)
        y = y + bl_ref[0, 0]
        y = y + jnp.dot(h_ref[...], wr_ref[...],
                        preferred_element_type=jnp.float32)
        y_ref[...] = y

    return pl.pallas_call(
        body,
        grid=(_GRID,),
        in_specs=[
            pl.BlockSpec((1, 2, _BN, w), lambda i: (0, 0, i, 0)),
            pl.BlockSpec((_BN, c), lambda i: (i, 0)),
            pl.BlockSpec((c, 1), lambda i: (0, 0)),
            pl.BlockSpec((1, 1), lambda i: (0, 0)),
            pl.BlockSpec((c, 1), lambda i: (0, 0)),
        ],
        out_specs=pl.BlockSpec((_BN, 1), lambda i: (i, 0)),
        out_shape=jax.ShapeDtypeStruct((_N_PAD, 1), jnp.float32),
    )


def kernel(x, edge_index, params, sage):
    n = x.shape[0]
    x0p = jnp.pad(x, ((0, _N_PAD - n), (0, 0)))
    src = edge_index[0]
    dst = edge_index[1]
    pad_idx = jnp.full((_E_PAD - src.shape[0],), n, jnp.int32)
    srcp = jnp.concatenate([src, pad_idx]).reshape(_NW, _N_CHUNKS, _CHUNK)
    dstp = jnp.concatenate([dst, pad_idx]).reshape(_NW, _N_CHUNKS, _CHUNK)

    h = x0p
    smax = _tc_max0(5)(x0p, params[0]['t'].reshape(1, 1))
    for li, p in enumerate(params):
        c, cout = _DIMS[li]
        nblk, w = _LAYER_BLOCKS[li]
        t = p['t'].reshape(1, 1)
        tabs = _tc_table(c, nblk, w)(h, smax, t)
        if not isinstance(tabs, (list, tuple)):
            tabs = [tabs]
        acc = _sc_agg(nblk, w)(srcp, dstp, *tabs)
        tn = (params[li + 1]['t'].reshape(1, 1) if li + 1 < len(params)
              else jnp.ones((1, 1), jnp.float32))
        hdim = 2 * c
        h, smax = _tc_post(c, cout, nblk, w)(
            acc, h, x0p,
            p['W1'], p['b1'].reshape(1, hdim), p['gamma'].reshape(1, hdim),
            p['beta'].reshape(1, hdim), p['W2'], p['b2'].reshape(1, cout), tn)

    tab = _tc_sage_table(13, 16)(h)
    acc = _sc_agg(1, 16)(srcp, dstp, tab)
    y = _tc_final(13, 16)(acc, h, sage['Wl'], sage['bl'].reshape(1, 1),
                          sage['Wr'])
    return y[:n]


# speedup vs baseline: 10.3926x; 1.0235x over previous
import functools

import jax
import jax.numpy as jnp
from jax import lax
from jax.experimental import pallas as pl
from jax.experimental.pallas import tpu as pltpu
from jax.experimental.pallas import tpu_sc as plsc

_EPS = 1e-7
_N = 50000
_E = 800000
_N_PAD = 51200
_NW = 32
_CHUNK = 128
_N_CHUNKS = 200
_E_PAD = _NW * _N_CHUNKS * _CHUNK
_RING = 8
_ROWS_PER_TILE = _N_PAD // 16
_Z_ROWS = 640
_BN = 1024
_GRID = _N_PAD // _BN

_LAYER_BLOCKS = [(1, 16), (2, 16), (3, 16), (5, 16), (9, 16), (5, 16), (3, 16)]
_DIMS = [(5, 8), (13, 16), (21, 32), (37, 64), (69, 32), (37, 16), (21, 8)]


@functools.cache
def _sc_agg(nblk, w):
    mesh = plsc.VectorSubcoreMesh(core_axis_name="c", subcore_axis_name="s")

    def body(src_hbm, dst_hbm, *rest):
        tables = rest[:nblk]
        out_hbm = rest[nblk]
        idx_s, idx_d, rows, zbuf, acc = rest[nblk + 1:nblk + 6]
        sg = rest[nblk + 6:nblk + 6 + _RING]
        ss = rest[nblk + 6 + _RING:nblk + 6 + 2 * _RING]
        sz = rest[nblk + 6 + 2 * _RING]
        cid = lax.axis_index("c")
        sid = lax.axis_index("s")
        wid = cid * 16 + sid
        pltpu.sync_copy(src_hbm.at[wid], idx_s)
        pltpu.sync_copy(dst_hbm.at[wid], idx_d)
        z16 = jnp.zeros((16,), jnp.float32)

        def zrow(r, carry):
            for c in range(w // 16):
                zbuf[r, pl.ds(c * 16, 16)] = z16
            return carry

        lax.fori_loop(0, _Z_ROWS, zrow, 0)
        rbase = sid * _ROWS_PER_TILE
        n_z = _ROWS_PER_TILE // _Z_ROWS

        def zero_acc():
            for j in range(n_z):
                pltpu.async_copy(
                    zbuf, acc.at[pl.ds(rbase + j * _Z_ROWS, _Z_ROWS)], sz)
            for j in range(n_z):
                pltpu.make_async_copy(
                    zbuf, acc.at[pl.ds(rbase + j * _Z_ROWS, _Z_ROWS)],
                    sz).wait()

        zero_acc()
        for b in range(nblk):
            tab = tables[b]
            plsc.subcore_barrier()
            for r in range(4):
                pltpu.async_copy(tab.at[idx_s.at[r]], rows.at[r], sg[r])

            def group(j, carry):
                for r in range(_RING):
                    i = _RING * j + r
                    rn = (r + 4) % _RING
                    pltpu.make_async_copy(tab.at[idx_s.at[i]], rows.at[r],
                                          sg[r]).wait()
                    pltpu.async_copy(rows.at[r], acc.at[idx_d.at[i]], ss[r],
                                     add=True)

                    @pl.when(i >= 4)
                    def _():
                        pltpu.make_async_copy(rows.at[rn],
                                              acc.at[idx_d.at[i - 4]],
                                              ss[rn]).wait()

                    @pl.when(i + 4 < _N_CHUNKS)
                    def _():
                        pltpu.async_copy(tab.at[idx_s.at[i + 4]], rows.at[rn],
                                         sg[rn])
                return carry

            lax.fori_loop(0, _N_CHUNKS // _RING, group, 0)
            for r in range(4, 8):
                i = _N_CHUNKS - 8 + r
                pltpu.make_async_copy(rows.at[r], acc.at[idx_d.at[i]],
                                      ss[r]).wait()
            plsc.subcore_barrier()
            pltpu.sync_copy(
                acc.at[pl.ds(rbase, _ROWS_PER_TILE)],
                out_hbm.at[b, cid, pl.ds(rbase, _ROWS_PER_TILE)])
            if b + 1 < nblk:
                zero_acc()

    return pl.kernel(
        body,
        out_type=jax.ShapeDtypeStruct((nblk, 2, _N_PAD, w), jnp.float32),
        mesh=mesh,
        scratch_types=(
            [
                pltpu.VMEM((_N_CHUNKS, _CHUNK), jnp.int32),
                pltpu.VMEM((_N_CHUNKS, _CHUNK), jnp.int32),
                pltpu.VMEM((_RING, _CHUNK, w), jnp.float32),
                pltpu.VMEM((_Z_ROWS, w), jnp.float32),
                pltpu.VMEM_SHARED((_N_PAD, w), jnp.float32),
            ]
            + [pltpu.SemaphoreType.DMA] * (2 * _RING + 1)
        ),
        compiler_params=pltpu.CompilerParams(use_tc_tiling_on_sc=False),
        name=f"sc_edge_agg_{nblk}x{w}",
    )


@functools.cache
def _tc_max0(c):
    def body(h_ref, t_ref, smax_ref):
        s = t_ref[0, 0] * (jnp.maximum(h_ref[...], 0.0) + _EPS)
        local = jnp.max(s, axis=0, keepdims=True)
        i = pl.program_id(0)

        @pl.when(i == 0)
        def _():
            smax_ref[...] = local

        @pl.when(i > 0)
        def _():
            smax_ref[...] = jnp.maximum(smax_ref[...], local)

    return pl.pallas_call(
        body,
        grid=(_GRID,),
        in_specs=[
            pl.BlockSpec((_BN, c), lambda i: (i, 0)),
            pl.BlockSpec((1, 1), lambda i: (0, 0)),
        ],
        out_specs=pl.BlockSpec((1, c), lambda i: (0, 0)),
        out_shape=jax.ShapeDtypeStruct((1, c), jnp.float32),
    )


@functools.cache
def _tc_table(c, nblk, w):
    def body(h_ref, smax_ref, t_ref, *out_refs):
        h = h_ref[...]
        msg = jnp.maximum(h, 0.0) + _EPS
        s = t_ref[0, 0] * msg
        e = jnp.exp(s - smax_ref[...])
        full = jnp.concatenate([e, e * msg], axis=1)
        padw = nblk * w - 2 * c
        if padw:
            full = jnp.concatenate(
                [full, jnp.zeros((_BN, padw), jnp.float32)], axis=1)
        for b in range(nblk):
            out_refs[b][...] = full[:, b * w:(b + 1) * w]

    return pl.pallas_call(
        body,
        grid=(_GRID,),
        in_specs=[
            pl.BlockSpec((_BN, c), lambda i: (i, 0)),
            pl.BlockSpec((1, c), lambda i: (0, 0)),
            pl.BlockSpec((1, 1), lambda i: (0, 0)),
        ],
        out_specs=[pl.BlockSpec((_BN, w), lambda i: (i, 0))] * nblk,
        out_shape=[jax.ShapeDtypeStruct((_N_PAD, w), jnp.float32)] * nblk,
    )


@functools.cache
def _tc_post(c, cout, nblk, w):
    hdim = 2 * c
    cn = cout + 5

    def body(acc_ref, h_ref, x0_ref, w1_ref, b1_ref, g_ref, be_ref, w2_ref,
             b2_ref, tn_ref, hcat_ref, smax_ref):
        blocks = [acc_ref[b, 0] + acc_ref[b, 1] for b in range(nblk)]
        full = jnp.concatenate(blocks, axis=1) if nblk > 1 else blocks[0]
        den = full[:, :c]
        num = full[:, c:2 * c]
        agg = num / jnp.maximum(den, 1e-16)
        out = agg + h_ref[...]
        hm = jnp.dot(out, w1_ref[...], preferred_element_type=jnp.float32)
        hm = (hm + b1_ref[...]) * g_ref[...] + be_ref[...]
        hm = jnp.maximum(hm, 0.0)
        h2 = jnp.dot(hm, w2_ref[...], preferred_element_type=jnp.float32)
        h2 = h2 + b2_ref[...]
        hcat = jnp.concatenate([h2, x0_ref[...]], axis=1)
        i = pl.program_id(0)
        rows = i * _BN + lax.broadcasted_iota(jnp.int32, (_BN, 1), 0)
        hcat = jnp.where(rows < _N, hcat, 0.0)
        hcat_ref[...] = hcat
        s = tn_ref[0, 0] * (jnp.maximum(hcat, 0.0) + _EPS)
        local = jnp.max(s, axis=0, keepdims=True)

        @pl.when(i == 0)
        def _():
            smax_ref[...] = local

        @pl.when(i > 0)
        def _():
            smax_ref[...] = jnp.maximum(smax_ref[...], local)

    return pl.pallas_call(
        body,
        grid=(_GRID,),
        in_specs=[
            pl.BlockSpec((nblk, 2, _BN, w), lambda i: (0, 0, i, 0)),
            pl.BlockSpec((_BN, c), lambda i: (i, 0)),
            pl.BlockSpec((_BN, 5), lambda i: (i, 0)),
            pl.BlockSpec((c, hdim), lambda i: (0, 0)),
            pl.BlockSpec((1, hdim), lambda i: (0, 0)),
            pl.BlockSpec((1, hdim), lambda i: (0, 0)),
            pl.BlockSpec((1, hdim), lambda i: (0, 0)),
            pl.BlockSpec((hdim, cout), lambda i: (0, 0)),
            pl.BlockSpec((1, cout), lambda i: (0, 0)),
            pl.BlockSpec((1, 1), lambda i: (0, 0)),
        ],
        out_specs=[
            pl.BlockSpec((_BN, cn), lambda i: (i, 0)),
            pl.BlockSpec((1, cn), lambda i: (0, 0)),
        ],
        out_shape=[
            jax.ShapeDtypeStruct((_N_PAD, cn), jnp.float32),
            jax.ShapeDtypeStruct((1, cn), jnp.float32),
        ],
    )


@functools.cache
def _tc_sage_table(c, w):
    def body(h_ref, out_ref):
        h = h_ref[...]
        ones = jnp.ones((_BN, 1), jnp.float32)
        pad = jnp.zeros((_BN, w - c - 1), jnp.float32)
        out_ref[...] = jnp.concatenate([h, ones, pad], axis=1)

    return pl.pallas_call(
        body,
        grid=(_GRID,),
        in_specs=[pl.BlockSpec((_BN, c), lambda i: (i, 0))],
        out_specs=pl.BlockSpec((_BN, w), lambda i: (i, 0)),
        out_shape=jax.ShapeDtypeStruct((_N_PAD, w), jnp.float32),
    )


@functools.cache
def _tc_final(c, w):
    def body(acc_ref, h_ref, wl_ref, bl_ref, wr_ref, y_ref):
        a = acc_ref[0, 0] + acc_ref[0, 1]
        ssum = a[:, :c]
        deg = a[:, c:c + 1]
        mean = ssum / jnp.maximum(deg, 1.0)
        y = jnp.dot(mean, wl_ref[...], preferred_element_type=jnp.float32)
        y = y + bl_ref[0, 0]
        y = y + jnp.dot(h_ref[...], wr_ref[...],
                        preferred_element_type=jnp.float32)
        y_ref[...] = y

    return pl.pallas_call(
        body,
        grid=(_GRID,),
        in_specs=[
            pl.BlockSpec((1, 2, _BN, w), lambda i: (0, 0, i, 0)),
            pl.BlockSpec((_BN, c), lambda i: (i, 0)),
            pl.BlockSpec((c, 1), lambda i: (0, 0)),
            pl.BlockSpec((1, 1), lambda i: (0, 0)),
            pl.BlockSpec((c, 1), lambda i: (0, 0)),
        ],
        out_specs=pl.BlockSpec((_BN, 1), lambda i: (i, 0)),
        out_shape=jax.ShapeDtypeStruct((_N_PAD, 1), jnp.float32),
    )


def kernel(x, edge_index, params, sage):
    n = x.shape[0]
    x0p = jnp.pad(x, ((0, _N_PAD - n), (0, 0)))
    src = edge_index[0]
    dst = edge_index[1]
    pad_idx = jnp.full((_E_PAD - src.shape[0],), n, jnp.int32)
    srcp = jnp.concatenate([src, pad_idx]).reshape(_NW, _N_CHUNKS, _CHUNK)
    dstp = jnp.concatenate([dst, pad_idx]).reshape(_NW, _N_CHUNKS, _CHUNK)

    h = x0p
    smax = _tc_max0(5)(x0p, params[0]['t'].reshape(1, 1))
    for li, p in enumerate(params):
        c, cout = _DIMS[li]
        nblk, w = _LAYER_BLOCKS[li]
        t = p['t'].reshape(1, 1)
        tabs = _tc_table(c, nblk, w)(h, smax, t)
        if not isinstance(tabs, (list, tuple)):
            tabs = [tabs]
        acc = _sc_agg(nblk, w)(srcp, dstp, *tabs)
        tn = (params[li + 1]['t'].reshape(1, 1) if li + 1 < len(params)
              else jnp.ones((1, 1), jnp.float32))
        hdim = 2 * c
        h, smax = _tc_post(c, cout, nblk, w)(
            acc, h, x0p,
            p['W1'], p['b1'].reshape(1, hdim), p['gamma'].reshape(1, hdim),
            p['beta'].reshape(1, hdim), p['W2'], p['b2'].reshape(1, cout), tn)

    tab = _tc_sage_table(13, 16)(h)
    acc = _sc_agg(1, 16)(srcp, dstp, tab)
    y = _tc_final(13, 16)(acc, h, sage['Wl'], sage['bl'].reshape(1, 1),
                          sage['Wr'])
    return y[:n]


# speedup vs baseline: 10.9909x; 1.0576x over previous
import functools

import jax
import jax.numpy as jnp
from jax import lax
from jax.experimental import pallas as pl
from jax.experimental.pallas import tpu as pltpu
from jax.experimental.pallas import tpu_sc as plsc

_EPS = 1e-7
_N = 50000
_E = 800000
_N_PAD = 51200
_NW = 32
_CHUNK = 128
_N_CHUNKS = 200
_E_PAD = _NW * _N_CHUNKS * _CHUNK
_RING = 8
_ROWS_PER_TILE = _N_PAD // 16
_Z_ROWS = 640
_BN = 1024
_GRID = _N_PAD // _BN

_LAYER_BLOCKS = [(1, 16), (2, 16), (3, 16), (5, 16), (9, 16), (5, 16), (3, 16)]
_DIMS = [(5, 8), (13, 16), (21, 32), (37, 64), (69, 32), (37, 16), (21, 8)]


@functools.cache
def _sc_agg(nblk, w):
    mesh = plsc.VectorSubcoreMesh(core_axis_name="c", subcore_axis_name="s")

    def body(src_hbm, dst_hbm, *rest):
        tables = rest[:nblk]
        out_hbm = rest[nblk]
        idx_s, idx_d, rows, zbuf, acc = rest[nblk + 1:nblk + 6]
        sg = rest[nblk + 6:nblk + 6 + _RING]
        ss = rest[nblk + 6 + _RING:nblk + 6 + 2 * _RING]
        sz = rest[nblk + 6 + 2 * _RING]
        cid = lax.axis_index("c")
        sid = lax.axis_index("s")
        wid = cid * 16 + sid
        pltpu.sync_copy(src_hbm.at[wid], idx_s)
        pltpu.sync_copy(dst_hbm.at[wid], idx_d)
        z16 = jnp.zeros((16,), jnp.float32)

        def zrow(r, carry):
            for c in range(w // 16):
                zbuf[r, pl.ds(c * 16, 16)] = z16
            return carry

        lax.fori_loop(0, _Z_ROWS, zrow, 0)
        rbase = sid * _ROWS_PER_TILE
        n_z = _ROWS_PER_TILE // _Z_ROWS

        def zero_acc():
            for j in range(n_z):
                pltpu.async_copy(
                    zbuf, acc.at[pl.ds(rbase + j * _Z_ROWS, _Z_ROWS)], sz)
            for j in range(n_z):
                pltpu.make_async_copy(
                    zbuf, acc.at[pl.ds(rbase + j * _Z_ROWS, _Z_ROWS)],
                    sz).wait()

        zero_acc()
        for b in range(nblk):
            tab = tables[b]
            plsc.subcore_barrier()
            for r in range(4):
                pltpu.async_copy(tab.at[idx_s.at[r]], rows.at[r], sg[r])

            def group(j, carry):
                for r in range(_RING):
                    i = _RING * j + r
                    rn = (r + 4) % _RING
                    pltpu.make_async_copy(tab.at[idx_s.at[i]], rows.at[r],
                                          sg[r]).wait()
                    pltpu.async_copy(rows.at[r], acc.at[idx_d.at[i]], ss[r],
                                     add=True)

                    @pl.when(i >= 4)
                    def _():
                        pltpu.make_async_copy(rows.at[rn],
                                              acc.at[idx_d.at[i - 4]],
                                              ss[rn]).wait()

                    @pl.when(i + 4 < _N_CHUNKS)
                    def _():
                        pltpu.async_copy(tab.at[idx_s.at[i + 4]], rows.at[rn],
                                         sg[rn])
                return carry

            lax.fori_loop(0, _N_CHUNKS // _RING, group, 0)
            for r in range(4, 8):
                i = _N_CHUNKS - 8 + r
                pltpu.make_async_copy(rows.at[r], acc.at[idx_d.at[i]],
                                      ss[r]).wait()
            plsc.subcore_barrier()
            pltpu.sync_copy(
                acc.at[pl.ds(rbase, _ROWS_PER_TILE)],
                out_hbm.at[b, cid, pl.ds(rbase, _ROWS_PER_TILE)])
            if b + 1 < nblk:
                zero_acc()

    return pl.kernel(
        body,
        out_type=jax.ShapeDtypeStruct((nblk, 2, _N_PAD, w), jnp.float32),
        mesh=mesh,
        scratch_types=(
            [
                pltpu.VMEM((_N_CHUNKS, _CHUNK), jnp.int32),
                pltpu.VMEM((_N_CHUNKS, _CHUNK), jnp.int32),
                pltpu.VMEM((_RING, _CHUNK, w), jnp.float32),
                pltpu.VMEM((_Z_ROWS, w), jnp.float32),
                pltpu.VMEM_SHARED((_N_PAD, w), jnp.float32),
            ]
            + [pltpu.SemaphoreType.DMA] * (2 * _RING + 1)
        ),
        compiler_params=pltpu.CompilerParams(use_tc_tiling_on_sc=False),
        name=f"sc_edge_agg_{nblk}x{w}",
    )


_SHIFT = 20.0


def _exp_table(h, t, nblk, w, c):
    msg = jnp.maximum(h, 0.0) + _EPS
    e = jnp.exp(t * msg - _SHIFT)
    full = jnp.concatenate([e, e * msg], axis=1)
    padw = nblk * w - 2 * c
    if padw:
        full = jnp.concatenate(
            [full, jnp.zeros((h.shape[0], padw), jnp.float32)], axis=1)
    return full


@functools.cache
def _tc_table0(c, nblk, w):
    def body(h_ref, t_ref, *out_refs):
        full = _exp_table(h_ref[...], t_ref[0, 0], nblk, w, c)
        for b in range(nblk):
            out_refs[b][...] = full[:, b * w:(b + 1) * w]

    return pl.pallas_call(
        body,
        grid=(_GRID,),
        in_specs=[
            pl.BlockSpec((_BN, c), lambda i: (i, 0)),
            pl.BlockSpec((1, 1), lambda i: (0, 0)),
        ],
        out_specs=[pl.BlockSpec((_BN, w), lambda i: (i, 0))] * nblk,
        out_shape=[jax.ShapeDtypeStruct((_N_PAD, w), jnp.float32)] * nblk,
    )


@functools.cache
def _tc_post(c, cout, nblk, w, nblk_out, w_out, sage):
    hdim = 2 * c
    cn = cout + 5

    def body(acc_ref, h_ref, x0_ref, w1_ref, b1_ref, g_ref, be_ref, w2_ref,
             b2_ref, tn_ref, hcat_ref, *tab_refs):
        blocks = [acc_ref[b, 0] + acc_ref[b, 1] for b in range(nblk)]
        full = jnp.concatenate(blocks, axis=1) if nblk > 1 else blocks[0]
        den = full[:, :c]
        num = full[:, c:2 * c]
        agg = num / jnp.maximum(den, 1e-16)
        out = agg + h_ref[...]
        hm = jnp.dot(out, w1_ref[...], preferred_element_type=jnp.float32)
        hm = (hm + b1_ref[...]) * g_ref[...] + be_ref[...]
        hm = jnp.maximum(hm, 0.0)
        h2 = jnp.dot(hm, w2_ref[...], preferred_element_type=jnp.float32)
        h2 = h2 + b2_ref[...]
        hcat = jnp.concatenate([h2, x0_ref[...]], axis=1)
        i = pl.program_id(0)
        rows = i * _BN + lax.broadcasted_iota(jnp.int32, (_BN, 1), 0)
        hcat = jnp.where(rows < _N, hcat, 0.0)
        hcat_ref[...] = hcat
        if sage:
            tful = jnp.concatenate(
                [hcat, jnp.ones((_BN, 1), jnp.float32),
                 jnp.zeros((_BN, w_out - cn - 1), jnp.float32)], axis=1)
        else:
            tful = _exp_table(hcat, tn_ref[0, 0], nblk_out, w_out, cn)
        for b in range(nblk_out):
            tab_refs[b][...] = tful[:, b * w_out:(b + 1) * w_out]

    return pl.pallas_call(
        body,
        grid=(_GRID,),
        in_specs=[
            pl.BlockSpec((nblk, 2, _BN, w), lambda i: (0, 0, i, 0)),
            pl.BlockSpec((_BN, c), lambda i: (i, 0)),
            pl.BlockSpec((_BN, 5), lambda i: (i, 0)),
            pl.BlockSpec((c, hdim), lambda i: (0, 0)),
            pl.BlockSpec((1, hdim), lambda i: (0, 0)),
            pl.BlockSpec((1, hdim), lambda i: (0, 0)),
            pl.BlockSpec((1, hdim), lambda i: (0, 0)),
            pl.BlockSpec((hdim, cout), lambda i: (0, 0)),
            pl.BlockSpec((1, cout), lambda i: (0, 0)),
            pl.BlockSpec((1, 1), lambda i: (0, 0)),
        ],
        out_specs=(
            [pl.BlockSpec((_BN, cn), lambda i: (i, 0))]
            + [pl.BlockSpec((_BN, w_out), lambda i: (i, 0))] * nblk_out
        ),
        out_shape=(
            [jax.ShapeDtypeStruct((_N_PAD, cn), jnp.float32)]
            + [jax.ShapeDtypeStruct((_N_PAD, w_out), jnp.float32)] * nblk_out
        ),
    )


@functools.cache
def _tc_final(c, w):
    def body(acc_ref, h_ref, wl_ref, bl_ref, wr_ref, y_ref):
        a = acc_ref[0, 0] + acc_ref[0, 1]
        ssum = a[:, :c]
        deg = a[:, c:c + 1]
        mean = ssum / jnp.maximum(deg, 1.0)
        y = jnp.dot(mean, wl_ref[...], preferred_element_type=jnp.float32)
        y = y + bl_ref[0, 0]
        y = y + jnp.dot(h_ref[...], wr_ref[...],
                        preferred_element_type=jnp.float32)
        y_ref[...] = y

    return pl.pallas_call(
        body,
        grid=(_GRID,),
        in_specs=[
            pl.BlockSpec((1, 2, _BN, w), lambda i: (0, 0, i, 0)),
            pl.BlockSpec((_BN, c), lambda i: (i, 0)),
            pl.BlockSpec((c, 1), lambda i: (0, 0)),
            pl.BlockSpec((1, 1), lambda i: (0, 0)),
            pl.BlockSpec((c, 1), lambda i: (0, 0)),
        ],
        out_specs=pl.BlockSpec((_BN, 1), lambda i: (i, 0)),
        out_shape=jax.ShapeDtypeStruct((_N_PAD, 1), jnp.float32),
    )


def kernel(x, edge_index, params, sage):
    n = x.shape[0]
    x0p = jnp.pad(x, ((0, _N_PAD - n), (0, 0)))
    src = edge_index[0]
    dst = edge_index[1]
    pad_idx = jnp.full((_E_PAD - src.shape[0],), n, jnp.int32)
    srcp = jnp.concatenate([src, pad_idx]).reshape(_NW, _N_CHUNKS, _CHUNK)
    dstp = jnp.concatenate([dst, pad_idx]).reshape(_NW, _N_CHUNKS, _CHUNK)

    h = x0p
    nblk0, w0 = _LAYER_BLOCKS[0]
    tabs = _tc_table0(5, nblk0, w0)(x0p, params[0]['t'].reshape(1, 1))
    if not isinstance(tabs, (list, tuple)):
        tabs = [tabs]
    for li, p in enumerate(params):
        c, cout = _DIMS[li]
        nblk, w = _LAYER_BLOCKS[li]
        acc = _sc_agg(nblk, w)(srcp, dstp, *tabs)
        last = li + 1 == len(params)
        nblk_out, w_out = (1, 16) if last else _LAYER_BLOCKS[li + 1]
        tn = (jnp.ones((1, 1), jnp.float32) if last
              else params[li + 1]['t'].reshape(1, 1))
        hdim = 2 * c
        outs = _tc_post(c, cout, nblk, w, nblk_out, w_out, last)(
            acc, h, x0p,
            p['W1'], p['b1'].reshape(1, hdim), p['gamma'].reshape(1, hdim),
            p['beta'].reshape(1, hdim), p['W2'], p['b2'].reshape(1, cout), tn)
        h = outs[0]
        tabs = outs[1:]

    acc = _sc_agg(1, 16)(srcp, dstp, *tabs)
    y = _tc_final(13, 16)(acc, h, sage['Wl'], sage['bl'].reshape(1, 1),
                          sage['Wr'])
    return y[:n]


# speedup vs baseline: 11.0135x; 1.0021x over previous
import functools

import jax
import jax.numpy as jnp
from jax import lax
from jax.experimental import pallas as pl
from jax.experimental.pallas import tpu as pltpu
from jax.experimental.pallas import tpu_sc as plsc

_EPS = 1e-7
_N = 50000
_E = 800000
_N_PAD = 51200
_NW = 32
_CHUNK = 128
_N_CHUNKS = 200
_E_PAD = _NW * _N_CHUNKS * _CHUNK
_RING = 8
_ROWS_PER_TILE = _N_PAD // 16
_Z_ROWS = 640
_BN = 1024
_GRID = _N_PAD // _BN

_LAYER_BLOCKS = [(1, 16), (2, 16), (3, 16), (5, 16), (9, 16), (5, 16), (3, 16)]
_DIMS = [(5, 8), (13, 16), (21, 32), (37, 64), (69, 32), (37, 16), (21, 8)]


@functools.cache
def _sc_agg(nblk, w):
    mesh = plsc.VectorSubcoreMesh(core_axis_name="c", subcore_axis_name="s")

    def body(src_hbm, dst_hbm, *rest):
        tables = rest[:nblk]
        out_hbm = rest[nblk]
        idx_s, idx_d, rows, zbuf, acc = rest[nblk + 1:nblk + 6]
        sg = rest[nblk + 6:nblk + 6 + _RING]
        ss = rest[nblk + 6 + _RING:nblk + 6 + 2 * _RING]
        sz = rest[nblk + 6 + 2 * _RING]
        cid = lax.axis_index("c")
        sid = lax.axis_index("s")
        wid = cid * 16 + sid
        pltpu.sync_copy(src_hbm.at[wid], idx_s)
        pltpu.sync_copy(dst_hbm.at[wid], idx_d)
        z16 = jnp.zeros((16,), jnp.float32)

        def zrow(r, carry):
            for c in range(w // 16):
                zbuf[r, pl.ds(c * 16, 16)] = z16
            return carry

        lax.fori_loop(0, _Z_ROWS, zrow, 0)
        rbase = sid * _ROWS_PER_TILE
        n_z = _ROWS_PER_TILE // _Z_ROWS

        def zero_acc():
            for j in range(n_z):
                pltpu.async_copy(
                    zbuf, acc.at[pl.ds(rbase + j * _Z_ROWS, _Z_ROWS)], sz)
            for j in range(n_z):
                pltpu.make_async_copy(
                    zbuf, acc.at[pl.ds(rbase + j * _Z_ROWS, _Z_ROWS)],
                    sz).wait()

        zero_acc()
        for r in range(4):
            pltpu.async_copy(tables[0].at[idx_s.at[r]], rows.at[r], sg[r])
        for b in range(nblk):
            tab = tables[b]
            plsc.subcore_barrier()

            def group(j, carry):
                for r in range(_RING):
                    i = _RING * j + r
                    rn = (r + 4) % _RING
                    pltpu.make_async_copy(tab.at[idx_s.at[i]], rows.at[r],
                                          sg[r]).wait()
                    pltpu.async_copy(rows.at[r], acc.at[idx_d.at[i]], ss[r],
                                     add=True)

                    @pl.when(i >= 4)
                    def _():
                        pltpu.make_async_copy(rows.at[rn],
                                              acc.at[idx_d.at[i - 4]],
                                              ss[rn]).wait()

                    @pl.when(i + 4 < _N_CHUNKS)
                    def _():
                        pltpu.async_copy(tab.at[idx_s.at[i + 4]], rows.at[rn],
                                         sg[rn])
                return carry

            lax.fori_loop(0, _N_CHUNKS // _RING, group, 0)
            for r in range(4, 8):
                i = _N_CHUNKS - 8 + r
                pltpu.make_async_copy(rows.at[r], acc.at[idx_d.at[i]],
                                      ss[r]).wait()
            plsc.subcore_barrier()
            if b + 1 < nblk:
                for r in range(4):
                    pltpu.async_copy(tables[b + 1].at[idx_s.at[r]],
                                     rows.at[r], sg[r])
            pltpu.sync_copy(
                acc.at[pl.ds(rbase, _ROWS_PER_TILE)],
                out_hbm.at[b, cid, pl.ds(rbase, _ROWS_PER_TILE)])
            if b + 1 < nblk:
                zero_acc()

    return pl.kernel(
        body,
        out_type=jax.ShapeDtypeStruct((nblk, 2, _N_PAD, w), jnp.float32),
        mesh=mesh,
        scratch_types=(
            [
                pltpu.VMEM((_N_CHUNKS, _CHUNK), jnp.int32),
                pltpu.VMEM((_N_CHUNKS, _CHUNK), jnp.int32),
                pltpu.VMEM((_RING, _CHUNK, w), jnp.float32),
                pltpu.VMEM((_Z_ROWS, w), jnp.float32),
                pltpu.VMEM_SHARED((_N_PAD, w), jnp.float32),
            ]
            + [pltpu.SemaphoreType.DMA] * (2 * _RING + 1)
        ),
        compiler_params=pltpu.CompilerParams(use_tc_tiling_on_sc=False),
        name=f"sc_edge_agg_{nblk}x{w}",
    )


_SHIFT = 20.0


def _exp_table(h, t, nblk, w, c):
    msg = jnp.maximum(h, 0.0) + _EPS
    e = jnp.exp(t * msg - _SHIFT)
    full = jnp.concatenate([e, e * msg], axis=1)
    padw = nblk * w - 2 * c
    if padw:
        full = jnp.concatenate(
            [full, jnp.zeros((h.shape[0], padw), jnp.float32)], axis=1)
    return full


@functools.cache
def _tc_table0(c, nblk, w):
    def body(h_ref, t_ref, *out_refs):
        full = _exp_table(h_ref[...], t_ref[0, 0], nblk, w, c)
        for b in range(nblk):
            out_refs[b][...] = full[:, b * w:(b + 1) * w]

    return pl.pallas_call(
        body,
        grid=(_GRID,),
        in_specs=[
            pl.BlockSpec((_BN, c), lambda i: (i, 0)),
            pl.BlockSpec((1, 1), lambda i: (0, 0)),
        ],
        out_specs=[pl.BlockSpec((_BN, w), lambda i: (i, 0))] * nblk,
        out_shape=[jax.ShapeDtypeStruct((_N_PAD, w), jnp.float32)] * nblk,
    )


@functools.cache
def _tc_post(c, cout, nblk, w, nblk_out, w_out, sage):
    hdim = 2 * c
    cn = cout + 5

    def body(acc_ref, h_ref, x0_ref, w1_ref, b1_ref, g_ref, be_ref, w2_ref,
             b2_ref, tn_ref, hcat_ref, *tab_refs):
        blocks = [acc_ref[b, 0] + acc_ref[b, 1] for b in range(nblk)]
        full = jnp.concatenate(blocks, axis=1) if nblk > 1 else blocks[0]
        den = full[:, :c]
        num = full[:, c:2 * c]
        agg = num / jnp.maximum(den, 1e-16)
        out = agg + h_ref[...]
        hm = jnp.dot(out, w1_ref[...], preferred_element_type=jnp.float32)
        hm = (hm + b1_ref[...]) * g_ref[...] + be_ref[...]
        hm = jnp.maximum(hm, 0.0)
        h2 = jnp.dot(hm, w2_ref[...], preferred_element_type=jnp.float32)
        h2 = h2 + b2_ref[...]
        hcat = jnp.concatenate([h2, x0_ref[...]], axis=1)
        i = pl.program_id(0)
        rows = i * _BN + lax.broadcasted_iota(jnp.int32, (_BN, 1), 0)
        hcat = jnp.where(rows < _N, hcat, 0.0)
        hcat_ref[...] = hcat
        if sage:
            tful = jnp.concatenate(
                [hcat, jnp.ones((_BN, 1), jnp.float32),
                 jnp.zeros((_BN, w_out - cn - 1), jnp.float32)], axis=1)
        else:
            tful = _exp_table(hcat, tn_ref[0, 0], nblk_out, w_out, cn)
        for b in range(nblk_out):
            tab_refs[b][...] = tful[:, b * w_out:(b + 1) * w_out]

    return pl.pallas_call(
        body,
        grid=(_GRID,),
        in_specs=[
            pl.BlockSpec((nblk, 2, _BN, w), lambda i: (0, 0, i, 0)),
            pl.BlockSpec((_BN, c), lambda i: (i, 0)),
            pl.BlockSpec((_BN, 5), lambda i: (i, 0)),
            pl.BlockSpec((c, hdim), lambda i: (0, 0)),
            pl.BlockSpec((1, hdim), lambda i: (0, 0)),
            pl.BlockSpec((1, hdim), lambda i: (0, 0)),
            pl.BlockSpec((1, hdim), lambda i: (0, 0)),
            pl.BlockSpec((hdim, cout), lambda i: (0, 0)),
            pl.BlockSpec((1, cout), lambda i: (0, 0)),
            pl.BlockSpec((1, 1), lambda i: (0, 0)),
        ],
        out_specs=(
            [pl.BlockSpec((_BN, cn), lambda i: (i, 0))]
            + [pl.BlockSpec((_BN, w_out), lambda i: (i, 0))] * nblk_out
        ),
        out_shape=(
            [jax.ShapeDtypeStruct((_N_PAD, cn), jnp.float32)]
            + [jax.ShapeDtypeStruct((_N_PAD, w_out), jnp.float32)] * nblk_out
        ),
    )


@functools.cache
def _tc_final(c, w):
    def body(acc_ref, h_ref, wl_ref, bl_ref, wr_ref, y_ref):
        a = acc_ref[0, 0] + acc_ref[0, 1]
        ssum = a[:, :c]
        deg = a[:, c:c + 1]
        mean = ssum / jnp.maximum(deg, 1.0)
        y = jnp.dot(mean, wl_ref[...], preferred_element_type=jnp.float32)
        y = y + bl_ref[0, 0]
        y = y + jnp.dot(h_ref[...], wr_ref[...],
                        preferred_element_type=jnp.float32)
        y_ref[...] = y

    return pl.pallas_call(
        body,
        grid=(_GRID,),
        in_specs=[
            pl.BlockSpec((1, 2, _BN, w), lambda i: (0, 0, i, 0)),
            pl.BlockSpec((_BN, c), lambda i: (i, 0)),
            pl.BlockSpec((c, 1), lambda i: (0, 0)),
            pl.BlockSpec((1, 1), lambda i: (0, 0)),
            pl.BlockSpec((c, 1), lambda i: (0, 0)),
        ],
        out_specs=pl.BlockSpec((_BN, 1), lambda i: (i, 0)),
        out_shape=jax.ShapeDtypeStruct((_N_PAD, 1), jnp.float32),
    )


def kernel(x, edge_index, params, sage):
    n = x.shape[0]
    x0p = jnp.pad(x, ((0, _N_PAD - n), (0, 0)))
    src = edge_index[0]
    dst = edge_index[1]
    pad_idx = jnp.full((_E_PAD - src.shape[0],), n, jnp.int32)
    srcp = jnp.concatenate([src, pad_idx]).reshape(_NW, _N_CHUNKS, _CHUNK)
    dstp = jnp.concatenate([dst, pad_idx]).reshape(_NW, _N_CHUNKS, _CHUNK)

    h = x0p
    nblk0, w0 = _LAYER_BLOCKS[0]
    tabs = _tc_table0(5, nblk0, w0)(x0p, params[0]['t'].reshape(1, 1))
    if not isinstance(tabs, (list, tuple)):
        tabs = [tabs]
    for li, p in enumerate(params):
        c, cout = _DIMS[li]
        nblk, w = _LAYER_BLOCKS[li]
        acc = _sc_agg(nblk, w)(srcp, dstp, *tabs)
        last = li + 1 == len(params)
        nblk_out, w_out = (1, 16) if last else _LAYER_BLOCKS[li + 1]
        tn = (jnp.ones((1, 1), jnp.float32) if last
              else params[li + 1]['t'].reshape(1, 1))
        hdim = 2 * c
        outs = _tc_post(c, cout, nblk, w, nblk_out, w_out, last)(
            acc, h, x0p,
            p['W1'], p['b1'].reshape(1, hdim), p['gamma'].reshape(1, hdim),
            p['beta'].reshape(1, hdim), p['W2'], p['b2'].reshape(1, cout), tn)
        h = outs[0]
        tabs = outs[1:]

    acc = _sc_agg(1, 16)(srcp, dstp, *tabs)
    y = _tc_final(13, 16)(acc, h, sage['Wl'], sage['bl'].reshape(1, 1),
                          sage['Wr'])
    return y[:n]
